# baseline (speedup 1.0000x reference)
import jax
import jax.numpy as jnp
from jax import lax
from jax.experimental import pallas as pl
from jax.experimental.pallas import tpu as pltpu
from jax.experimental.pallas import tpu_sc as plsc

N = 10000
E = 320000
H0 = 128
OUT = 256
LAT = 64
NT = 10
NB = 55
SH = 16
RMAX = 5.0
XW = NT + SH

BE = 6400
BN = 2000

SC_NC = 2
SC_NS = 16
SC_NW = SC_NC * SC_NS
SC_CH = E // SC_NW


def _edge_body(x_ref, h0_ref, bt_ref, wl_ref, wc_ref, wp_ref, be_ref, me_ref,
               lat_ref, ef_ref):
    x = x_ref[...]
    lat_ref[...] = jnp.dot(x, wl_ref[...], preferred_element_type=jnp.float32)
    bti = bt_ref[...].astype(jnp.int32)
    sel = (bti == jax.lax.broadcasted_iota(jnp.int32, (BE, NB), 1)
           ).astype(jnp.bfloat16)
    mrow = jnp.dot(sel, me_ref[...], preferred_element_type=jnp.float32)
    src = (h0_ref[...] * mrow).astype(jnp.bfloat16)
    ef = jnp.dot(x, wc_ref[...], preferred_element_type=jnp.float32)
    ef = ef + jnp.dot(src, wp_ref[...], preferred_element_type=jnp.float32)
    ef_ref[...] = ef + be_ref[...]


def _node_body(xn_ref, h0_ref, emb_ref, mn_ref, wn_ref, bn_ref, nf_ref):
    ati = xn_ref[...][:, 0:1].astype(jnp.int32)
    sel = (ati == jax.lax.broadcasted_iota(jnp.int32, (BN, NT), 1)
           ).astype(jnp.float32)
    base = jnp.dot(sel, emb_ref[...], preferred_element_type=jnp.float32)
    mrow = jnp.dot(sel, mn_ref[...], preferred_element_type=jnp.float32)
    src = h0_ref[...] * mrow
    nf = base + jnp.dot(src, wn_ref[...], preferred_element_type=jnp.float32)
    nf_ref[...] = nf + bn_ref[...]


def _sc_cutoff_body(el_hbm, co_hbm, ae_hbm, el_v, co_v, ae_v):
    wid = lax.axis_index("s") * SC_NC + lax.axis_index("c")
    base = wid * SC_CH
    pltpu.sync_copy(el_hbm.at[pl.ds(base, SC_CH)], el_v)

    def body(i, carry):
        el = el_v[pl.ds(i * 16, 16)]
        x = jnp.clip(el * (1.0 / RMAX), 0.0, 1.0)
        t = jnp.float32(jnp.pi) * (x - 0.5)
        t2 = t * t
        s = t * (1.0 + t2 * (-1.0 / 6.0 + t2 * (1.0 / 120.0 + t2 * (
            -1.0 / 5040.0 + t2 * (1.0 / 362880.0)))))
        co_v[pl.ds(i * 16, 16)] = 0.5 * (1.0 - s)
        ae_v[pl.ds(i * 16, 16)] = base + i * 16 + lax.iota(jnp.int32, 16)
        return carry

    lax.fori_loop(0, SC_CH // 16, body, 0)
    pltpu.sync_copy(co_v, co_hbm.at[pl.ds(base, SC_CH)])
    pltpu.sync_copy(ae_v, ae_hbm.at[pl.ds(base, SC_CH)])


def kernel(node_h0, edge_h0, edge_index, atom_type, bond_type, edge_sh,
           edge_length, edge_one_hot, W_latent, W_edge_base, atom_embed,
           W_node_proj, b_node, W_edge_proj, b_edge, mask_nrme, mask_erme):
    x = jnp.concatenate(
        [edge_one_hot.astype(jnp.bfloat16), edge_sh.astype(jnp.bfloat16)],
        axis=1)
    bt8 = bond_type.astype(jnp.int8)[:, None]
    xn = atom_type.astype(jnp.float32)[:, None]
    wl = W_latent.astype(jnp.bfloat16)
    wc = (W_latent @ W_edge_base).astype(jnp.bfloat16)
    wp = W_edge_proj.astype(jnp.bfloat16)
    me = mask_erme.astype(jnp.bfloat16)
    be2 = b_edge.reshape(1, OUT)
    bn2 = b_node.reshape(1, OUT)

    row = lambda i: (i, 0)
    full = lambda i: (0, 0)

    lat, ef = pl.pallas_call(
        _edge_body,
        grid=(E // BE,),
        in_specs=[
            pl.BlockSpec((BE, XW), row),
            pl.BlockSpec((BE, H0), row),
            pl.BlockSpec((BE, 1), row),
            pl.BlockSpec((XW, LAT), full),
            pl.BlockSpec((XW, OUT), full),
            pl.BlockSpec((H0, OUT), full),
            pl.BlockSpec((1, OUT), full),
            pl.BlockSpec((NB, H0), full),
        ],
        out_specs=[
            pl.BlockSpec((BE, LAT), row),
            pl.BlockSpec((BE, OUT), row),
        ],
        out_shape=[
            jax.ShapeDtypeStruct((E, LAT), jnp.float32),
            jax.ShapeDtypeStruct((E, OUT), jnp.float32),
        ],
    )(x, edge_h0, bt8, wl, wc, wp, be2, me)

    co, ae = pl.kernel(
        _sc_cutoff_body,
        out_type=[
            jax.ShapeDtypeStruct((E,), jnp.float32),
            jax.ShapeDtypeStruct((E,), jnp.int32),
        ],
        mesh=plsc.VectorSubcoreMesh(core_axis_name="c", subcore_axis_name="s",
                                    num_cores=SC_NC, num_subcores=SC_NS),
        scratch_types=[
            pltpu.VMEM((SC_CH,), jnp.float32),
            pltpu.VMEM((SC_CH,), jnp.float32),
            pltpu.VMEM((SC_CH,), jnp.int32),
        ],
    )(edge_length)

    nf = pl.pallas_call(
        _node_body,
        grid=(N // BN,),
        in_specs=[
            pl.BlockSpec((BN, 1), row),
            pl.BlockSpec((BN, H0), row),
            pl.BlockSpec((NT, OUT), full),
            pl.BlockSpec((NT, H0), full),
            pl.BlockSpec((H0, OUT), full),
            pl.BlockSpec((1, OUT), full),
        ],
        out_specs=pl.BlockSpec((BN, OUT), row),
        out_shape=jax.ShapeDtypeStruct((N, OUT), jnp.float32),
    )(xn, node_h0, atom_embed, mask_nrme, W_node_proj, bn2)

    return (lat, nf, ef, co, ae)


# speedup vs baseline: 1.3615x; 1.3615x over previous
import jax
import jax.numpy as jnp
from jax import lax
from jax.experimental import pallas as pl
from jax.experimental.pallas import tpu as pltpu
from jax.experimental.pallas import tpu_sc as plsc

N = 10000
E = 320000
H0 = 128
OUT = 256
LAT = 64
NT = 10
NB = 55
SH = 16
RMAX = 5.0
XW = NT + SH + 1

BE = 6400
BN = 2000

SC_NC = 2
SC_NS = 16
SC_NW = SC_NC * SC_NS
SC_CH = E // SC_NW


def _edge_body(x_ref, h0_ref, wl_ref, wc_ref, wp_ref, be_ref, me_ref,
               lat_ref, ef_ref):
    x = x_ref[...]
    lat_ref[...] = jnp.dot(x, wl_ref[...], preferred_element_type=jnp.float32)
    bti = x[:, XW - 1:XW].astype(jnp.int32)
    sel = (bti == jax.lax.broadcasted_iota(jnp.int32, (BE, NB), 1)
           ).astype(jnp.bfloat16)
    mrow = jnp.dot(sel, me_ref[...], preferred_element_type=jnp.float32)
    src = (h0_ref[...] * mrow).astype(jnp.bfloat16)
    ef = jnp.dot(x, wc_ref[...], preferred_element_type=jnp.float32)
    ef = ef + jnp.dot(src, wp_ref[...], preferred_element_type=jnp.float32)
    ef_ref[...] = ef + be_ref[...]


def _node_body(xn_ref, h0_ref, emb_ref, mn_ref, wn_ref, bn_ref, nf_ref):
    ati = xn_ref[...][:, 0:1].astype(jnp.int32)
    sel = (ati == jax.lax.broadcasted_iota(jnp.int32, (BN, NT), 1)
           ).astype(jnp.float32)
    base = jnp.dot(sel, emb_ref[...], preferred_element_type=jnp.float32)
    mrow = jnp.dot(sel, mn_ref[...], preferred_element_type=jnp.float32)
    src = h0_ref[...] * mrow
    nf = base + jnp.dot(src, wn_ref[...], preferred_element_type=jnp.float32)
    nf_ref[...] = nf + bn_ref[...]


def _tc_cutoff_body(el_ref, co_ref, ae_ref):
    xx = jnp.clip(el_ref[...] * (1.0 / RMAX), 0.0, 1.0)
    co_ref[...] = 0.5 * (jnp.cos(jnp.pi * xx) + 1.0)
    rows, cols = ae_ref.shape
    ae_ref[...] = (jax.lax.broadcasted_iota(jnp.int32, (rows, cols), 0) * cols
                   + jax.lax.broadcasted_iota(jnp.int32, (rows, cols), 1))


def _sc_cutoff_body(el_hbm, co_hbm, ae_hbm, el_v, co_v, ae_v):
    wid = lax.axis_index("s") * SC_NC + lax.axis_index("c")
    base = wid * SC_CH
    pltpu.sync_copy(el_hbm.at[pl.ds(base, SC_CH)], el_v)

    def body(i, carry):
        el = el_v[pl.ds(i * 16, 16)]
        x = jnp.clip(el * (1.0 / RMAX), 0.0, 1.0)
        t = jnp.float32(jnp.pi) * (x - 0.5)
        t2 = t * t
        s = t * (1.0 + t2 * (-1.0 / 6.0 + t2 * (1.0 / 120.0 + t2 * (
            -1.0 / 5040.0 + t2 * (1.0 / 362880.0)))))
        co_v[pl.ds(i * 16, 16)] = 0.5 * (1.0 - s)
        ae_v[pl.ds(i * 16, 16)] = base + i * 16 + lax.iota(jnp.int32, 16)
        return carry

    lax.fori_loop(0, SC_CH // 16, body, 0)
    pltpu.sync_copy(co_v, co_hbm.at[pl.ds(base, SC_CH)])
    pltpu.sync_copy(ae_v, ae_hbm.at[pl.ds(base, SC_CH)])


def kernel(node_h0, edge_h0, edge_index, atom_type, bond_type, edge_sh,
           edge_length, edge_one_hot, W_latent, W_edge_base, atom_embed,
           W_node_proj, b_node, W_edge_proj, b_edge, mask_nrme, mask_erme):
    x = jnp.concatenate(
        [edge_one_hot.astype(jnp.bfloat16), edge_sh.astype(jnp.bfloat16),
         bond_type.astype(jnp.bfloat16)[:, None]], axis=1)
    xn = atom_type.astype(jnp.float32)[:, None]
    wlf = jnp.concatenate([W_latent, jnp.zeros((1, LAT), jnp.float32)], axis=0)
    wl = wlf.astype(jnp.bfloat16)
    wc = (wlf @ W_edge_base).astype(jnp.bfloat16)
    wp = W_edge_proj.astype(jnp.bfloat16)
    me = mask_erme.astype(jnp.bfloat16)
    be2 = b_edge.reshape(1, OUT)
    bn2 = b_node.reshape(1, OUT)

    row = lambda i: (i, 0)
    full = lambda i: (0, 0)

    lat, ef = pl.pallas_call(
        _edge_body,
        grid=(E // BE,),
        in_specs=[
            pl.BlockSpec((BE, XW), row),
            pl.BlockSpec((BE, H0), row),
            pl.BlockSpec((XW, LAT), full),
            pl.BlockSpec((XW, OUT), full),
            pl.BlockSpec((H0, OUT), full),
            pl.BlockSpec((1, OUT), full),
            pl.BlockSpec((NB, H0), full),
        ],
        out_specs=[
            pl.BlockSpec((BE, LAT), row),
            pl.BlockSpec((BE, OUT), row),
        ],
        out_shape=[
            jax.ShapeDtypeStruct((E, LAT), jnp.float32),
            jax.ShapeDtypeStruct((E, OUT), jnp.float32),
        ],
    )(x, edge_h0, wl, wc, wp, be2, me)

    el2 = edge_length.reshape(E // 128, 128)
    co2, ae2 = pl.pallas_call(
        _tc_cutoff_body,
        grid=(1,),
        in_specs=[pl.BlockSpec((E // 128, 128), full)],
        out_specs=[
            pl.BlockSpec((E // 128, 128), full),
            pl.BlockSpec((E // 128, 128), full),
        ],
        out_shape=[
            jax.ShapeDtypeStruct((E // 128, 128), jnp.float32),
            jax.ShapeDtypeStruct((E // 128, 128), jnp.int32),
        ],
    )(el2)
    co, ae = co2.reshape(E), ae2.reshape(E)

    nf = pl.pallas_call(
        _node_body,
        grid=(N // BN,),
        in_specs=[
            pl.BlockSpec((BN, 1), row),
            pl.BlockSpec((BN, H0), row),
            pl.BlockSpec((NT, OUT), full),
            pl.BlockSpec((NT, H0), full),
            pl.BlockSpec((H0, OUT), full),
            pl.BlockSpec((1, OUT), full),
        ],
        out_specs=pl.BlockSpec((BN, OUT), row),
        out_shape=jax.ShapeDtypeStruct((N, OUT), jnp.float32),
    )(xn, node_h0, atom_embed, mask_nrme, W_node_proj, bn2)

    return (lat, nf, ef, co, ae)
